# baseline (speedup 1.0000x reference)
import jax
import jax.numpy as jnp
from jax.experimental import pallas as pl

_DIMS = [(3, 60), (60, 60), (60, 60), (60, 60), (60, 120), (120, 120),
         (120, 120), (120, 150), (150, 200), (200, 210), (210, 250),
         (250, 300), (300, 300), (300, 300), (300, 300), (300, 300)]
_JOINT = 512


def _elu(x):
    return jnp.where(x > 0, x, jnp.exp(jnp.minimum(x, 0.0)) - 1.0)


def _encoder_body(*refs):
    pos_ref, adj_ref = refs[0], refs[1]
    wrefs = refs[2:-1]
    out_ref = refs[-1]

    adj = adj_ref[0]
    norm = jnp.sum(adj, axis=1, keepdims=True, dtype=jnp.float32)
    inv_norm = 1.0 / norm

    x = pos_ref[0]
    for i, (fi, fo) in enumerate(_DIMS):
        w = wrefs[2 * i][...]
        b = wrefs[2 * i + 1][...]
        support = jnp.dot(x, w, preferred_element_type=jnp.float32)
        s = max(fo // 3, 2)
        c = min(fo, 128)
        col_c = jax.lax.broadcasted_iota(jnp.int32, (1, c), 1)
        pre = jnp.where(col_c < s, support[:, :c] * inv_norm, 0.0)
        side = jnp.dot(adj, pre.astype(jnp.bfloat16),
                       preferred_element_type=jnp.float32)
        if fo > c:
            side = jnp.concatenate(
                [side, jnp.zeros((side.shape[0], fo - c), side.dtype)], axis=1)
        col_f = jax.lax.broadcasted_iota(jnp.int32, (1, fo), 1)
        out = jnp.where(col_f < s, side, support) + b
        x = _elu(out)

    wr = wrefs[-2][...]
    br = wrefs[-1][...]
    support = jnp.dot(x, wr, preferred_element_type=jnp.float32)
    out = jnp.dot(adj, support.astype(jnp.bfloat16),
                  preferred_element_type=jnp.float32) + br
    latent = jnp.max(out, axis=0, keepdims=True)
    out_ref[...] = _elu(latent).reshape(1, 1, _JOINT)


def kernel(positions, adj,
           W0, b0, W1, b1, W2, b2, W3, b3,
           W4, b4, W5, b5, W6, b6, W7, b7,
           W8, b8, W9, b9, W10, b10, W11, b11,
           W12, b12, W13, b13, W14, b14, W15, b15,
           Wr, br):
    B, N, _ = positions.shape
    ws = [W0, b0, W1, b1, W2, b2, W3, b3, W4, b4, W5, b5, W6, b6, W7, b7,
          W8, b8, W9, b9, W10, b10, W11, b11, W12, b12, W13, b13, W14, b14,
          W15, b15]

    args = [positions, adj.astype(jnp.bfloat16)]
    in_specs = [
        pl.BlockSpec((1, N, 3), lambda b: (b, 0, 0)),
        pl.BlockSpec((1, N, N), lambda b: (b, 0, 0)),
    ]
    for i, (fi, fo) in enumerate(_DIMS):
        args.append(ws[2 * i])
        in_specs.append(pl.BlockSpec((fi, fo), lambda b: (0, 0)))
        args.append(ws[2 * i + 1].reshape(1, fo))
        in_specs.append(pl.BlockSpec((1, fo), lambda b: (0, 0)))
    args.append(Wr)
    in_specs.append(pl.BlockSpec(Wr.shape, lambda b: (0, 0)))
    args.append(br.reshape(1, _JOINT))
    in_specs.append(pl.BlockSpec((1, _JOINT), lambda b: (0, 0)))

    out = pl.pallas_call(
        _encoder_body,
        grid=(B,),
        in_specs=in_specs,
        out_specs=pl.BlockSpec((1, 1, _JOINT), lambda b: (b, 0, 0)),
        out_shape=jax.ShapeDtypeStruct((B, 1, _JOINT), jnp.float32),
    )(*args)
    return out.reshape(B, _JOINT)


# speedup vs baseline: 1.0856x; 1.0856x over previous
import jax
import jax.numpy as jnp
from jax.experimental import pallas as pl
from jax.experimental.pallas import tpu as pltpu

_DIMS = [(3, 60), (60, 60), (60, 60), (60, 60), (60, 120), (120, 120),
         (120, 120), (120, 150), (150, 200), (200, 210), (210, 250),
         (250, 300), (300, 300), (300, 300), (300, 300), (300, 300)]
_JOINT = 512


def _elu(x):
    return jnp.where(x > 0, x, jnp.exp(jnp.minimum(x, 0.0)) - 1.0)


def _encoder_body(*refs):
    pos_ref, adj_ref = refs[0], refs[1]
    wrefs = refs[2:-1]
    out_ref = refs[-1]

    adj = adj_ref[0]
    norm = jnp.sum(adj, axis=1, keepdims=True, dtype=jnp.float32)
    inv_norm = 1.0 / norm

    x = pos_ref[0]
    for i, (fi, fo) in enumerate(_DIMS):
        w = wrefs[2 * i][...]
        b = wrefs[2 * i + 1][...]
        support = jnp.dot(x, w, preferred_element_type=jnp.float32)
        s = max(fo // 3, 2)
        c = min(fo, 128)
        col_c = jax.lax.broadcasted_iota(jnp.int32, (1, c), 1)
        pre = jnp.where(col_c < s, support[:, :c] * inv_norm, 0.0)
        side = jnp.dot(adj, pre, preferred_element_type=jnp.float32)
        if fo > c:
            side = jnp.concatenate(
                [side, jnp.zeros((side.shape[0], fo - c), side.dtype)], axis=1)
        col_f = jax.lax.broadcasted_iota(jnp.int32, (1, fo), 1)
        out = jnp.where(col_f < s, side, support) + b
        x = _elu(out)

    wr = wrefs[-2][...]
    br = wrefs[-1][...]
    support = jnp.dot(x, wr, preferred_element_type=jnp.float32)
    out = jnp.dot(adj, support, preferred_element_type=jnp.float32) + br
    latent = jnp.max(out, axis=0, keepdims=True)
    out_ref[...] = _elu(latent).reshape(1, 1, _JOINT)


def kernel(positions, adj,
           W0, b0, W1, b1, W2, b2, W3, b3,
           W4, b4, W5, b5, W6, b6, W7, b7,
           W8, b8, W9, b9, W10, b10, W11, b11,
           W12, b12, W13, b13, W14, b14, W15, b15,
           Wr, br):
    B, N, _ = positions.shape
    ws = [W0, b0, W1, b1, W2, b2, W3, b3, W4, b4, W5, b5, W6, b6, W7, b7,
          W8, b8, W9, b9, W10, b10, W11, b11, W12, b12, W13, b13, W14, b14,
          W15, b15]

    args = [positions, adj]
    in_specs = [
        pl.BlockSpec((1, N, 3), lambda b: (b, 0, 0)),
        pl.BlockSpec((1, N, N), lambda b: (b, 0, 0)),
    ]
    for i, (fi, fo) in enumerate(_DIMS):
        args.append(ws[2 * i])
        in_specs.append(pl.BlockSpec((fi, fo), lambda b: (0, 0)))
        args.append(ws[2 * i + 1].reshape(1, fo))
        in_specs.append(pl.BlockSpec((1, fo), lambda b: (0, 0)))
    args.append(Wr)
    in_specs.append(pl.BlockSpec(Wr.shape, lambda b: (0, 0)))
    args.append(br.reshape(1, _JOINT))
    in_specs.append(pl.BlockSpec((1, _JOINT), lambda b: (0, 0)))

    out = pl.pallas_call(
        _encoder_body,
        grid=(B,),
        in_specs=in_specs,
        out_specs=pl.BlockSpec((1, 1, _JOINT), lambda b: (b, 0, 0)),
        out_shape=jax.ShapeDtypeStruct((B, 1, _JOINT), jnp.float32),
        compiler_params=pltpu.CompilerParams(
            dimension_semantics=("parallel",)),
    )(*args)
    return out.reshape(B, _JOINT)


# speedup vs baseline: 1.0900x; 1.0040x over previous
import jax
import jax.numpy as jnp
from jax.experimental import pallas as pl
from jax.experimental.pallas import tpu as pltpu

_DIMS = [(3, 60), (60, 60), (60, 60), (60, 60), (60, 120), (120, 120),
         (120, 120), (120, 150), (150, 200), (200, 210), (210, 250),
         (250, 300), (300, 300), (300, 300), (300, 300), (300, 300)]
_JOINT = 512


def _elu(x):
    return jnp.where(x > 0, x, jnp.exp(jnp.minimum(x, 0.0)) - 1.0)


def _encoder_body(*refs):
    pos_ref, adj_ref = refs[0], refs[1]
    wrefs = refs[2:-1]
    out_ref = refs[-1]

    adj = adj_ref[0]
    norm = jnp.sum(adj, axis=1, keepdims=True, dtype=jnp.float32)
    inv_norm = 1.0 / norm

    x = pos_ref[0]
    for i, (fi, fo) in enumerate(_DIMS):
        w = wrefs[2 * i][...]
        b = wrefs[2 * i + 1][...]
        support = jnp.dot(x, w, preferred_element_type=jnp.float32)
        s = max(fo // 3, 2)
        c = min(fo, 128)
        col_c = jax.lax.broadcasted_iota(jnp.int32, (1, c), 1)
        pre = jnp.where(col_c < s, support[:, :c] * inv_norm, 0.0)
        side = jnp.dot(adj, pre, preferred_element_type=jnp.float32)
        if fo > c:
            side = jnp.concatenate(
                [side, jnp.zeros((side.shape[0], fo - c), side.dtype)], axis=1)
        col_f = jax.lax.broadcasted_iota(jnp.int32, (1, fo), 1)
        out = jnp.where(col_f < s, side, support) + b
        x = _elu(out)

    wr = wrefs[-2][...]
    br = wrefs[-1][...]
    agg = jnp.dot(adj, x, preferred_element_type=jnp.float32)
    out = jnp.dot(agg, wr, preferred_element_type=jnp.float32) + br
    latent = jnp.max(out, axis=0, keepdims=True)
    out_ref[...] = _elu(latent).reshape(1, 1, _JOINT)


def kernel(positions, adj,
           W0, b0, W1, b1, W2, b2, W3, b3,
           W4, b4, W5, b5, W6, b6, W7, b7,
           W8, b8, W9, b9, W10, b10, W11, b11,
           W12, b12, W13, b13, W14, b14, W15, b15,
           Wr, br):
    B, N, _ = positions.shape
    ws = [W0, b0, W1, b1, W2, b2, W3, b3, W4, b4, W5, b5, W6, b6, W7, b7,
          W8, b8, W9, b9, W10, b10, W11, b11, W12, b12, W13, b13, W14, b14,
          W15, b15]

    args = [positions, adj]
    in_specs = [
        pl.BlockSpec((1, N, 3), lambda b: (b, 0, 0)),
        pl.BlockSpec((1, N, N), lambda b: (b, 0, 0)),
    ]
    for i, (fi, fo) in enumerate(_DIMS):
        args.append(ws[2 * i])
        in_specs.append(pl.BlockSpec((fi, fo), lambda b: (0, 0)))
        args.append(ws[2 * i + 1].reshape(1, fo))
        in_specs.append(pl.BlockSpec((1, fo), lambda b: (0, 0)))
    args.append(Wr)
    in_specs.append(pl.BlockSpec(Wr.shape, lambda b: (0, 0)))
    args.append(br.reshape(1, _JOINT))
    in_specs.append(pl.BlockSpec((1, _JOINT), lambda b: (0, 0)))

    out = pl.pallas_call(
        _encoder_body,
        grid=(B,),
        in_specs=in_specs,
        out_specs=pl.BlockSpec((1, 1, _JOINT), lambda b: (b, 0, 0)),
        out_shape=jax.ShapeDtypeStruct((B, 1, _JOINT), jnp.float32),
        compiler_params=pltpu.CompilerParams(
            dimension_semantics=("parallel",)),
    )(*args)
    return out.reshape(B, _JOINT)


# speedup vs baseline: 1.4951x; 1.3716x over previous
import jax
import jax.numpy as jnp
from jax.experimental import pallas as pl
from jax.experimental.pallas import tpu as pltpu

_DIMS = [(3, 60), (60, 60), (60, 60), (60, 60), (60, 120), (120, 120),
         (120, 120), (120, 150), (150, 200), (200, 210), (210, 250),
         (250, 300), (300, 300), (300, 300), (300, 300), (300, 300)]
_JOINT = 512


def _elu(x):
    return jnp.where(x > 0, x, jnp.exp(jnp.minimum(x, 0.0)) - 1.0)


def _encoder_body(*refs):
    pos_a, pos_b, adj_hbm = refs[0], refs[1], refs[2]
    wrefs = refs[3:-4]
    out_a, out_b = refs[-4], refs[-3]
    adj_vmem, dma_sem = refs[-2], refs[-1]

    g = pl.program_id(0)
    cp0 = pltpu.make_async_copy(adj_hbm.at[2 * g], adj_vmem.at[0], dma_sem)
    cp1 = pltpu.make_async_copy(adj_hbm.at[2 * g + 1], adj_vmem.at[1],
                                dma_sem)
    cp0.start()
    cp1.start()
    cp0.wait()
    cp1.wait()

    adjs = [adj_vmem[0], adj_vmem[1]]
    inv_norms = [
        1.0 / jnp.sum(a, axis=1, keepdims=True, dtype=jnp.float32)
        for a in adjs
    ]
    xs = [pos_a[0], pos_b[0]]

    for i, (fi, fo) in enumerate(_DIMS):
        w = wrefs[2 * i][...]
        b = wrefs[2 * i + 1][...]
        s = max(fo // 3, 2)
        c = min(fo, 128)
        col_c = jax.lax.broadcasted_iota(jnp.int32, (1, c), 1)
        col_f = jax.lax.broadcasted_iota(jnp.int32, (1, fo), 1)
        for j in range(2):
            support = jnp.dot(xs[j], w, preferred_element_type=jnp.float32)
            pre = jnp.where(col_c < s, support[:, :c] * inv_norms[j], 0.0)
            side = jnp.dot(adjs[j], pre.astype(jnp.bfloat16),
                           preferred_element_type=jnp.float32)
            if fo > c:
                side = jnp.concatenate(
                    [side, jnp.zeros((side.shape[0], fo - c), side.dtype)],
                    axis=1)
            xs[j] = _elu(jnp.where(col_f < s, side, support) + b)

    wr = wrefs[-2][...]
    br = wrefs[-1][...]
    for j, oref in enumerate((out_a, out_b)):
        agg = jnp.dot(adjs[j], xs[j].astype(jnp.bfloat16),
                      preferred_element_type=jnp.float32)
        out = jnp.dot(agg, wr, preferred_element_type=jnp.float32) + br
        latent = jnp.max(out, axis=0, keepdims=True)
        oref[...] = _elu(latent).reshape(1, 1, _JOINT)


def kernel(positions, adj,
           W0, b0, W1, b1, W2, b2, W3, b3,
           W4, b4, W5, b5, W6, b6, W7, b7,
           W8, b8, W9, b9, W10, b10, W11, b11,
           W12, b12, W13, b13, W14, b14, W15, b15,
           Wr, br):
    B, N, _ = positions.shape
    ws = [W0, b0, W1, b1, W2, b2, W3, b3, W4, b4, W5, b5, W6, b6, W7, b7,
          W8, b8, W9, b9, W10, b10, W11, b11, W12, b12, W13, b13, W14, b14,
          W15, b15]
    adj_bf = adj.astype(jnp.bfloat16)

    args = [positions, positions, adj_bf]
    in_specs = [
        pl.BlockSpec((1, N, 3), lambda g: (2 * g, 0, 0)),
        pl.BlockSpec((1, N, 3), lambda g: (2 * g + 1, 0, 0)),
        pl.BlockSpec(memory_space=pltpu.MemorySpace.HBM),
    ]
    for i, (fi, fo) in enumerate(_DIMS):
        args.append(ws[2 * i])
        in_specs.append(pl.BlockSpec((fi, fo), lambda g: (0, 0)))
        args.append(ws[2 * i + 1].reshape(1, fo))
        in_specs.append(pl.BlockSpec((1, fo), lambda g: (0, 0)))
    args.append(Wr)
    in_specs.append(pl.BlockSpec(Wr.shape, lambda g: (0, 0)))
    args.append(br.reshape(1, _JOINT))
    in_specs.append(pl.BlockSpec((1, _JOINT), lambda g: (0, 0)))

    half = B // 2
    out_a, out_b = pl.pallas_call(
        _encoder_body,
        grid=(half,),
        in_specs=in_specs,
        out_specs=[
            pl.BlockSpec((1, 1, _JOINT), lambda g: (g, 0, 0)),
            pl.BlockSpec((1, 1, _JOINT), lambda g: (g, 0, 0)),
        ],
        out_shape=[
            jax.ShapeDtypeStruct((half, 1, _JOINT), jnp.float32),
            jax.ShapeDtypeStruct((half, 1, _JOINT), jnp.float32),
        ],
        scratch_shapes=[
            pltpu.MemorySpace.VMEM((2, N, N), jnp.bfloat16),
            pltpu.SemaphoreType.DMA,
        ],
        compiler_params=pltpu.CompilerParams(
            dimension_semantics=("arbitrary",)),
    )(*args)
    return jnp.stack([out_a, out_b], axis=1).reshape(B, _JOINT)


# speedup vs baseline: 1.6361x; 1.0943x over previous
import jax
import jax.numpy as jnp
from jax.experimental import pallas as pl
from jax.experimental.pallas import tpu as pltpu

_DIMS = [(3, 60), (60, 60), (60, 60), (60, 60), (60, 120), (120, 120),
         (120, 120), (120, 150), (150, 200), (200, 210), (210, 250),
         (250, 300), (300, 300), (300, 300), (300, 300), (300, 300)]
_JOINT = 512


def _elu(x):
    return jnp.where(x > 0, x, jnp.exp(jnp.minimum(x, 0.0)) - 1.0)


def _encoder_body(*refs):
    pos_a, pos_b, adj_hbm = refs[0], refs[1], refs[2]
    wrefs = refs[3:-4]
    out_a, out_b = refs[-4], refs[-3]
    adj_vmem, dma_sem = refs[-2], refs[-1]

    g = pl.program_id(0)
    cp0 = pltpu.make_async_copy(adj_hbm.at[2 * g], adj_vmem.at[0], dma_sem)
    cp1 = pltpu.make_async_copy(adj_hbm.at[2 * g + 1], adj_vmem.at[1],
                                dma_sem)
    cp0.start()
    cp1.start()
    cp0.wait()
    cp1.wait()

    adjs = [adj_vmem[0], adj_vmem[1]]
    inv_norms = [
        1.0 / jnp.sum(a, axis=1, keepdims=True, dtype=jnp.float32)
        for a in adjs
    ]
    xs = [pos_a[0], pos_b[0]]

    for i, (fi, fo) in enumerate(_DIMS):
        w = wrefs[2 * i][...]
        b = wrefs[2 * i + 1][...]
        s = max(fo // 3, 2)
        c = min(fo, 128)
        col_c = jax.lax.broadcasted_iota(jnp.int32, (1, c), 1)
        for j in range(2):
            support = jnp.dot(xs[j], w, preferred_element_type=jnp.float32)
            pre = jnp.where(col_c < s, support[:, :c] * inv_norms[j], 0.0)
            side = jnp.dot(adjs[j], pre, preferred_element_type=jnp.float32)
            left = _elu(jnp.where(col_c < s, side, support[:, :c]) + b[:, :c])
            if fo > c:
                right = _elu(support[:, c:] + b[:, c:])
                xs[j] = jnp.concatenate([left, right], axis=1)
            else:
                xs[j] = left

    wr = wrefs[-2][...]
    br = wrefs[-1][...]
    for j, oref in enumerate((out_a, out_b)):
        agg = jnp.dot(adjs[j], xs[j], preferred_element_type=jnp.float32)
        out = jnp.dot(agg, wr, preferred_element_type=jnp.float32) + br
        latent = jnp.max(out, axis=0, keepdims=True)
        oref[...] = _elu(latent).reshape(1, 1, _JOINT)


def kernel(positions, adj,
           W0, b0, W1, b1, W2, b2, W3, b3,
           W4, b4, W5, b5, W6, b6, W7, b7,
           W8, b8, W9, b9, W10, b10, W11, b11,
           W12, b12, W13, b13, W14, b14, W15, b15,
           Wr, br):
    B, N, _ = positions.shape
    ws = [W0, b0, W1, b1, W2, b2, W3, b3, W4, b4, W5, b5, W6, b6, W7, b7,
          W8, b8, W9, b9, W10, b10, W11, b11, W12, b12, W13, b13, W14, b14,
          W15, b15]
    args = [positions, positions, adj]
    in_specs = [
        pl.BlockSpec((1, N, 3), lambda g: (2 * g, 0, 0)),
        pl.BlockSpec((1, N, 3), lambda g: (2 * g + 1, 0, 0)),
        pl.BlockSpec(memory_space=pltpu.MemorySpace.HBM),
    ]
    for i, (fi, fo) in enumerate(_DIMS):
        args.append(ws[2 * i])
        in_specs.append(pl.BlockSpec((fi, fo), lambda g: (0, 0)))
        args.append(ws[2 * i + 1].reshape(1, fo))
        in_specs.append(pl.BlockSpec((1, fo), lambda g: (0, 0)))
    args.append(Wr)
    in_specs.append(pl.BlockSpec(Wr.shape, lambda g: (0, 0)))
    args.append(br.reshape(1, _JOINT))
    in_specs.append(pl.BlockSpec((1, _JOINT), lambda g: (0, 0)))

    half = B // 2
    out_a, out_b = pl.pallas_call(
        _encoder_body,
        grid=(half,),
        in_specs=in_specs,
        out_specs=[
            pl.BlockSpec((1, 1, _JOINT), lambda g: (g, 0, 0)),
            pl.BlockSpec((1, 1, _JOINT), lambda g: (g, 0, 0)),
        ],
        out_shape=[
            jax.ShapeDtypeStruct((half, 1, _JOINT), jnp.float32),
            jax.ShapeDtypeStruct((half, 1, _JOINT), jnp.float32),
        ],
        scratch_shapes=[
            pltpu.MemorySpace.VMEM((2, N, N), jnp.float32),
            pltpu.SemaphoreType.DMA,
        ],
        compiler_params=pltpu.CompilerParams(
            dimension_semantics=("arbitrary",)),
    )(*args)
    return jnp.stack([out_a, out_b], axis=1).reshape(B, _JOINT)


# speedup vs baseline: 1.6436x; 1.0046x over previous
import jax
import jax.numpy as jnp
from jax.experimental import pallas as pl
from jax.experimental.pallas import tpu as pltpu

_DIMS = [(3, 60), (60, 60), (60, 60), (60, 60), (60, 120), (120, 120),
         (120, 120), (120, 150), (150, 200), (200, 210), (210, 250),
         (250, 300), (300, 300), (300, 300), (300, 300), (300, 300)]
_JOINT = 512


def _elu(x):
    return jnp.where(x > 0, x, jnp.exp(x) - 1.0)


def _encoder_body(*refs):
    pos_a, pos_b, adj_hbm = refs[0], refs[1], refs[2]
    wrefs = refs[3:-4]
    out_a, out_b = refs[-4], refs[-3]
    adj_vmem, dma_sem = refs[-2], refs[-1]

    g = pl.program_id(0)
    cp0 = pltpu.make_async_copy(adj_hbm.at[2 * g], adj_vmem.at[0], dma_sem)
    cp1 = pltpu.make_async_copy(adj_hbm.at[2 * g + 1], adj_vmem.at[1],
                                dma_sem)
    cp0.start()
    cp1.start()
    cp0.wait()
    cp1.wait()

    adjs = [adj_vmem[0], adj_vmem[1]]
    inv_norms = [
        1.0 / jnp.sum(a, axis=1, keepdims=True, dtype=jnp.float32)
        for a in adjs
    ]
    xs = [pos_a[0], pos_b[0]]

    for i, (fi, fo) in enumerate(_DIMS):
        w = wrefs[2 * i][...]
        b = wrefs[2 * i + 1][...]
        s = max(fo // 3, 2)
        c = min(fo, 128)
        col_c = jax.lax.broadcasted_iota(jnp.int32, (1, c), 1)
        for j in range(2):
            support = jnp.dot(xs[j], w, preferred_element_type=jnp.float32)
            pre = jnp.where(col_c < s, support[:, :c] * inv_norms[j], 0.0)
            side = jnp.dot(adjs[j], pre, preferred_element_type=jnp.float32)
            left = _elu(jnp.where(col_c < s, side, support[:, :c]) + b[:, :c])
            if fo > c:
                right = _elu(support[:, c:] + b[:, c:])
                xs[j] = jnp.concatenate([left, right], axis=1)
            else:
                xs[j] = left

    wr = wrefs[-2][...]
    br = wrefs[-1][...]
    for j, oref in enumerate((out_a, out_b)):
        agg = jnp.dot(adjs[j], xs[j], preferred_element_type=jnp.float32)
        out = jnp.dot(agg, wr, preferred_element_type=jnp.float32) + br
        latent = jnp.max(out, axis=0, keepdims=True)
        oref[...] = _elu(latent).reshape(1, 1, _JOINT)


def kernel(positions, adj,
           W0, b0, W1, b1, W2, b2, W3, b3,
           W4, b4, W5, b5, W6, b6, W7, b7,
           W8, b8, W9, b9, W10, b10, W11, b11,
           W12, b12, W13, b13, W14, b14, W15, b15,
           Wr, br):
    B, N, _ = positions.shape
    ws = [W0, b0, W1, b1, W2, b2, W3, b3, W4, b4, W5, b5, W6, b6, W7, b7,
          W8, b8, W9, b9, W10, b10, W11, b11, W12, b12, W13, b13, W14, b14,
          W15, b15]
    args = [positions, positions, adj]
    in_specs = [
        pl.BlockSpec((1, N, 3), lambda g: (2 * g, 0, 0)),
        pl.BlockSpec((1, N, 3), lambda g: (2 * g + 1, 0, 0)),
        pl.BlockSpec(memory_space=pltpu.MemorySpace.HBM),
    ]
    for i, (fi, fo) in enumerate(_DIMS):
        args.append(ws[2 * i])
        in_specs.append(pl.BlockSpec((fi, fo), lambda g: (0, 0)))
        args.append(ws[2 * i + 1].reshape(1, fo))
        in_specs.append(pl.BlockSpec((1, fo), lambda g: (0, 0)))
    args.append(Wr)
    in_specs.append(pl.BlockSpec(Wr.shape, lambda g: (0, 0)))
    args.append(br.reshape(1, _JOINT))
    in_specs.append(pl.BlockSpec((1, _JOINT), lambda g: (0, 0)))

    half = B // 2
    out_a, out_b = pl.pallas_call(
        _encoder_body,
        grid=(half,),
        in_specs=in_specs,
        out_specs=[
            pl.BlockSpec((1, 1, _JOINT), lambda g: (g, 0, 0)),
            pl.BlockSpec((1, 1, _JOINT), lambda g: (g, 0, 0)),
        ],
        out_shape=[
            jax.ShapeDtypeStruct((half, 1, _JOINT), jnp.float32),
            jax.ShapeDtypeStruct((half, 1, _JOINT), jnp.float32),
        ],
        scratch_shapes=[
            pltpu.MemorySpace.VMEM((2, N, N), jnp.float32),
            pltpu.SemaphoreType.DMA,
        ],
        compiler_params=pltpu.CompilerParams(
            dimension_semantics=("arbitrary",)),
    )(*args)
    return jnp.stack([out_a, out_b], axis=1).reshape(B, _JOINT)
